# Initial kernel scaffold; baseline (speedup 1.0000x reference)
import jax
import jax.numpy as jnp
from jax.experimental import pallas as pl


def kernel(x, edge_index, params):
    raise NotImplementedError("write your pallas kernel here")


# speedup vs baseline: 6.6470x; 6.6470x over previous
import functools

import jax
import jax.numpy as jnp
import numpy as np
from jax import lax
from jax.experimental import pallas as pl
from jax.experimental.pallas import tpu as pltpu
from jax.experimental.pallas import tpu_sc as plsc

N = 10000
E = 320000
D_IN = 128
H = 256
HH = H // 2
NUM_LAYERS = 4

NC, NS, LANES = 2, 16, 16
CHUNK = 128
C_HIST = 79
EPAD = NC * NS * C_HIST * CHUNK
C_SC = EPAD // (NS * CHUNK)
DUMMY = N
NPAD = 10240
RPS = NPAD // NS
BLK = 512
GRID = NPAD // BLK
BN_SCALE = float(1.0 / np.sqrt(1.0 + 1e-5))

_sc_mesh = plsc.VectorSubcoreMesh(
    core_axis_name="c", subcore_axis_name="s", num_cores=NC, num_subcores=NS)


@functools.partial(
    pl.kernel,
    out_type=jax.ShapeDtypeStruct((NC, NPAD, 16), jnp.float32),
    mesh=_sc_mesh,
    scratch_types=[
        pltpu.VMEM((CHUNK,), jnp.int32),
        pltpu.VMEM((CHUNK, 16), jnp.float32),
        pltpu.VMEM_SHARED((NPAD, 16), jnp.float32),
    ],
)
def _sc_hist(dst_hbm, ones_hbm, zeros_hbm, degp_hbm, didx, ones_v, acc):
    c = lax.axis_index("c")
    s = lax.axis_index("s")
    w = c * NS + s
    pltpu.sync_copy(zeros_hbm, acc.at[pl.ds(s * RPS, RPS)])
    pltpu.sync_copy(ones_hbm, ones_v)
    plsc.subcore_barrier()

    def chunk(j, carry):
        off = (w * C_HIST + j) * CHUNK
        pltpu.sync_copy(dst_hbm.at[pl.ds(off, CHUNK)], didx)
        pltpu.sync_copy(ones_v, acc.at[didx], add=True)
        return carry

    lax.fori_loop(0, C_HIST, chunk, 0)
    plsc.subcore_barrier()
    pltpu.sync_copy(acc.at[pl.ds(s * RPS, RPS)],
                    degp_hbm.at[c, pl.ds(s * RPS, RPS)])


@functools.partial(
    pl.kernel,
    out_type=jax.ShapeDtypeStruct((NC, NPAD, 128), jnp.float32),
    mesh=_sc_mesh,
    scratch_types=[
        pltpu.VMEM((CHUNK,), jnp.int32),
        pltpu.VMEM((CHUNK,), jnp.int32),
        pltpu.VMEM((CHUNK, 128), jnp.float32),
        pltpu.VMEM_SHARED((NPAD, 128), jnp.float32),
    ],
)
def _sc_scatter(src_hbm, dst_hbm, lp0_hbm, lp1_hbm, zeros_hbm, agg_hbm,
                sidx, didx, rows, acc):
    c = lax.axis_index("c")
    s = lax.axis_index("s")
    pltpu.sync_copy(zeros_hbm, acc.at[pl.ds(s * RPS, RPS)])
    plsc.subcore_barrier()

    def chunk(j, carry):
        off = (s * C_SC + j) * CHUNK
        pltpu.sync_copy(src_hbm.at[pl.ds(off, CHUNK)], sidx)
        pltpu.sync_copy(dst_hbm.at[pl.ds(off, CHUNK)], didx)

        @pl.when(c == 0)
        def _():
            pltpu.sync_copy(lp0_hbm.at[sidx], rows)

        @pl.when(c == 1)
        def _():
            pltpu.sync_copy(lp1_hbm.at[sidx], rows)

        pltpu.sync_copy(rows, acc.at[didx], add=True)
        return carry

    lax.fori_loop(0, C_SC, chunk, 0)
    plsc.subcore_barrier()
    pltpu.sync_copy(acc.at[pl.ds(s * RPS, RPS)],
                    agg_hbm.at[c, pl.ds(s * RPS, RPS)])


def _dinv(degp):
    deg = degp[0, :, 0:1] + degp[1, :, 0:1] + 1.0
    return lax.rsqrt(deg)


def _tc0_body(x_ref, degp_ref, w_ref, lp_ref):
    dinv = _dinv(degp_ref[...])
    lp = jnp.dot(x_ref[...], w_ref[...], preferred_element_type=jnp.float32)
    lp = lp * dinv
    lp_ref[0] = lp[:, :128]
    lp_ref[1] = lp[:, 128:]


_tc0 = pl.pallas_call(
    _tc0_body,
    grid=(GRID,),
    in_specs=[
        pl.BlockSpec((BLK, D_IN), lambda i: (i, 0)),
        pl.BlockSpec((2, BLK, 16), lambda i: (0, i, 0)),
        pl.BlockSpec((D_IN, H), lambda i: (0, 0)),
    ],
    out_specs=pl.BlockSpec((2, BLK, 128), lambda i: (0, i, 0)),
    out_shape=jax.ShapeDtypeStruct((2, NPAD, 128), jnp.float32),
)


def _tcmid_body(*refs, has_res):
    if has_res:
        (agg_ref, lpp_ref, hres_ref, degp_ref, w_ref, sc_ref, bi_ref,
         h_ref, lp_ref) = refs
    else:
        (agg_ref, lpp_ref, degp_ref, w_ref, sc_ref, bi_ref,
         h_ref, lp_ref) = refs
    dinv = _dinv(degp_ref[...])
    aggf = jnp.concatenate([agg_ref[0], agg_ref[1]], axis=1)
    lpp = jnp.concatenate([lpp_ref[0], lpp_ref[1]], axis=1)
    pre = (aggf + lpp) * dinv * sc_ref[...] + bi_ref[...]
    h = jnp.maximum(pre, 0.0)
    if has_res:
        h = h + hres_ref[...]
    h_ref[...] = h
    lp = jnp.dot(h, w_ref[...], preferred_element_type=jnp.float32) * dinv
    lp_ref[0] = lp[:, :128]
    lp_ref[1] = lp[:, 128:]


def _make_tcmid(has_res):
    specs = [
        pl.BlockSpec((2, BLK, 128), lambda i: (0, i, 0)),
        pl.BlockSpec((2, BLK, 128), lambda i: (0, i, 0)),
    ] + ([pl.BlockSpec((BLK, H), lambda i: (i, 0))] if has_res else []) + [
        pl.BlockSpec((2, BLK, 16), lambda i: (0, i, 0)),
        pl.BlockSpec((H, H), lambda i: (0, 0)),
        pl.BlockSpec((1, H), lambda i: (0, 0)),
        pl.BlockSpec((1, H), lambda i: (0, 0)),
    ]
    return pl.pallas_call(
        functools.partial(_tcmid_body, has_res=has_res),
        grid=(GRID,),
        in_specs=specs,
        out_specs=[
            pl.BlockSpec((BLK, H), lambda i: (i, 0)),
            pl.BlockSpec((2, BLK, 128), lambda i: (0, i, 0)),
        ],
        out_shape=[
            jax.ShapeDtypeStruct((NPAD, H), jnp.float32),
            jax.ShapeDtypeStruct((2, NPAD, 128), jnp.float32),
        ],
    )


_tcmid_nores = _make_tcmid(False)
_tcmid_res = _make_tcmid(True)


def _tc4_body(agg_ref, lpp_ref, hres_ref, degp_ref, sc_ref, bi_ref,
              aw1_ref, ab1_ref, aw2_ref, h4_ref, s_ref):
    dinv = _dinv(degp_ref[...])
    aggf = jnp.concatenate([agg_ref[0], agg_ref[1]], axis=1)
    lpp = jnp.concatenate([lpp_ref[0], lpp_ref[1]], axis=1)
    pre = (aggf + lpp) * dinv * sc_ref[...] + bi_ref[...]
    h4 = jnp.maximum(pre, 0.0) + hres_ref[...]
    h4_ref[...] = h4
    t = jnp.tanh(jnp.dot(h4, aw1_ref[...], preferred_element_type=jnp.float32)
                 + ab1_ref[...])
    sc = jnp.sum(t * aw2_ref[...], axis=1, keepdims=True)
    row = pl.program_id(0) * BLK + lax.broadcasted_iota(jnp.int32, (BLK, 1), 0)
    s_ref[...] = jnp.where(row < N, sc, -1e30)


_tc4 = pl.pallas_call(
    _tc4_body,
    grid=(GRID,),
    in_specs=[
        pl.BlockSpec((2, BLK, 128), lambda i: (0, i, 0)),
        pl.BlockSpec((2, BLK, 128), lambda i: (0, i, 0)),
        pl.BlockSpec((BLK, H), lambda i: (i, 0)),
        pl.BlockSpec((2, BLK, 16), lambda i: (0, i, 0)),
        pl.BlockSpec((1, H), lambda i: (0, 0)),
        pl.BlockSpec((1, H), lambda i: (0, 0)),
        pl.BlockSpec((H, HH), lambda i: (0, 0)),
        pl.BlockSpec((1, HH), lambda i: (0, 0)),
        pl.BlockSpec((1, HH), lambda i: (0, 0)),
    ],
    out_specs=[
        pl.BlockSpec((BLK, H), lambda i: (i, 0)),
        pl.BlockSpec((BLK, 1), lambda i: (i, 0)),
    ],
    out_shape=[
        jax.ShapeDtypeStruct((NPAD, H), jnp.float32),
        jax.ShapeDtypeStruct((NPAD, 1), jnp.float32),
    ],
)


def _tc5_body(h4_ref, s_ref, hw1_ref, hb1_ref, hw2_ref, hb2_ref,
              hw3_ref, hb3_ref, out_ref):
    s = s_ref[...]
    m = jnp.max(s)
    w = jnp.exp(s - m)
    z = jnp.sum(w)
    g = jnp.sum(h4_ref[...] * w, axis=0, keepdims=True) / z
    z1 = jnp.maximum(
        jnp.dot(g, hw1_ref[...], preferred_element_type=jnp.float32)
        + hb1_ref[...], 0.0)
    z2 = jnp.maximum(
        jnp.dot(z1, hw2_ref[...], preferred_element_type=jnp.float32)
        + hb2_ref[...], 0.0)
    out_ref[...] = (jnp.dot(z2, hw3_ref[...], preferred_element_type=jnp.float32)
                    + hb3_ref[...])


_tc5 = pl.pallas_call(
    _tc5_body,
    out_shape=jax.ShapeDtypeStruct((1, 2), jnp.float32),
)


def kernel(x, edge_index, params):
    src = edge_index[0]
    dst = edge_index[1]
    src_p = jnp.concatenate([src, jnp.zeros((EPAD - E,), jnp.int32)])
    dst_p = jnp.concatenate([dst, jnp.full((EPAD - E,), DUMMY, jnp.int32)])
    x_p = jnp.concatenate([x, jnp.zeros((NPAD - N, D_IN), jnp.float32)])
    zeros_h = jnp.zeros((RPS, 16), jnp.float32)
    zeros_s = jnp.zeros((RPS, 128), jnp.float32)
    ones_h = jnp.ones((CHUNK, 16), jnp.float32)

    degp = _sc_hist(dst_p, ones_h, zeros_h)

    lp = _tc0(x_p, degp, params["conv_W"][0])
    h = None
    for i in range(1, NUM_LAYERS):
        agg = _sc_scatter(src_p, dst_p, lp[0], lp[1], zeros_s)
        scale = (BN_SCALE * params["bn_gamma"][i - 1])[None, :]
        bias = (params["bn_beta"][i - 1]
                + params["conv_b"][i - 1] * BN_SCALE * params["bn_gamma"][i - 1])[None, :]
        if i == 1:
            h, lp = _tcmid_nores(agg, lp, degp, params["conv_W"][i], scale, bias)
        else:
            h, lp = _tcmid_res(agg, lp, h, degp, params["conv_W"][i], scale, bias)

    agg = _sc_scatter(src_p, dst_p, lp[0], lp[1], zeros_s)
    scale3 = (BN_SCALE * params["bn_gamma"][3])[None, :]
    bias3 = (params["bn_beta"][3]
             + params["conv_b"][3] * BN_SCALE * params["bn_gamma"][3])[None, :]
    h4, s = _tc4(agg, lp, h, degp, scale3, bias3,
                 params["att_W1"], params["att_b1"][None, :],
                 params["att_W2"][:, 0][None, :])
    preds = _tc5(h4, s,
                 params["head_W1"], params["head_b1"][None, :],
                 params["head_W2"], params["head_b2"][None, :],
                 params["head_W3"], params["head_b3"][None, :])
    return preds
